# Initial kernel scaffold; baseline (speedup 1.0000x reference)
import jax
import jax.numpy as jnp
from jax.experimental import pallas as pl


def kernel(x, edge_index, edge_weight, W1, W1_mask, b1, W2, W2_mask, b2):
    raise NotImplementedError("write your pallas kernel here")


# speedup vs baseline: 21.1159x; 21.1159x over previous
import functools

import jax
import jax.numpy as jnp
from jax import lax
from jax.experimental import pallas as pl
from jax.experimental.pallas import tpu as pltpu
from jax.experimental.pallas import tpu_sc as plsc

NC = 2
NS = 16
NW = NC * NS
CH = 128


def _deg_kernel(n_pad, nch):
    npt = n_pad // NS

    @functools.partial(
        pl.kernel,
        out_type=jax.ShapeDtypeStruct((NC, n_pad), jnp.float32),
        mesh=plsc.VectorSubcoreMesh(core_axis_name="c", subcore_axis_name="s"),
        scratch_types=[
            pltpu.VMEM((nch, CH), jnp.int32),
            pltpu.VMEM((nch, CH), jnp.float32),
            pltpu.VMEM((npt,), jnp.float32),
            pltpu.VMEM_SHARED((n_pad,), jnp.float32),
        ],
    )
    def k(dst_hbm, ew_hbm, out_hbm, dstv, ewv, zb, deg_sh):
        c = lax.axis_index("c")
        s = lax.axis_index("s")
        w = c * NS + s

        def zfill(i, _):
            zb[pl.ds(i * 16, 16)] = jnp.zeros((16,), jnp.float32)
            return 0

        lax.fori_loop(0, npt // 16, zfill, 0)
        pltpu.sync_copy(zb, deg_sh.at[pl.ds(s * npt, npt)])
        pltpu.sync_copy(dst_hbm.at[w], dstv)
        pltpu.sync_copy(ew_hbm.at[w], ewv)
        plsc.subcore_barrier()

        def chunk(j, _):
            pltpu.sync_copy(ewv.at[j], deg_sh.at[dstv.at[j]], add=True)
            return 0

        lax.fori_loop(0, nch, chunk, 0)
        plsc.subcore_barrier()
        pltpu.sync_copy(deg_sh.at[pl.ds(s * npt, npt)],
                        out_hbm.at[c, pl.ds(s * npt, npt)])

    return k


def _msg_kernel(n, n_pad, nch, h):
    npt = n_pad // NS
    zr = 64

    @functools.partial(
        pl.kernel,
        out_type=jax.ShapeDtypeStruct((NC, n_pad, h), jnp.float32),
        mesh=plsc.VectorSubcoreMesh(core_axis_name="c", subcore_axis_name="s"),
        scratch_types=[
            pltpu.VMEM((nch, CH), jnp.int32),
            pltpu.VMEM((nch, CH), jnp.int32),
            pltpu.VMEM((nch * CH,), jnp.float32),
            pltpu.VMEM((CH, h), jnp.float32),
            pltpu.VMEM((zr, h), jnp.float32),
            pltpu.VMEM_SHARED((n_pad, h), jnp.float32),
            pltpu.SemaphoreType.DMA,
        ],
        compiler_params=pltpu.CompilerParams(needs_layout_passes=False,
                                             use_tc_tiling_on_sc=False),
    )
    def k(src_hbm, dst_hbm, ew_hbm, h2_hbm, out_hbm,
          srcv, dstv, ewv, rowsv, zb, acc_sh, gsem):
        c = lax.axis_index("c")
        s = lax.axis_index("s")
        w = c * NS + s
        base = s * npt

        def zfill(i, _):
            for q in range(h // 16):
                zb[i, pl.ds(q * 16, 16)] = jnp.zeros((16,), jnp.float32)
            return 0

        lax.fori_loop(0, zr, zfill, 0)
        for i in range(npt // zr):
            pltpu.sync_copy(zb, acc_sh.at[pl.ds(base + i * zr, zr)])
        pltpu.sync_copy(src_hbm.at[w], srcv)
        pltpu.sync_copy(dst_hbm.at[w], dstv)
        pltpu.sync_copy(ew_hbm.at[w], ewv)
        plsc.subcore_barrier()

        def chunk(j, _):
            pltpu.async_copy(h2_hbm.at[srcv.at[j]], rowsv, gsem).wait()

            def grp(g, _):
                for kk in range(16):
                    le = g * 16 + kk
                    bi = jnp.full((16,), j * CH + le, jnp.int32)
                    wv = plsc.load_gather(ewv, [bi])
                    for q in range(h // 16):
                        sl = pl.ds(q * 16, 16)
                        rowsv[le, sl] = rowsv[le, sl] * wv
                return 0

            lax.fori_loop(0, CH // 16, grp, 0)
            pltpu.sync_copy(rowsv, acc_sh.at[dstv.at[j]], add=True)
            return 0

        lax.fori_loop(0, nch, chunk, 0)
        plsc.subcore_barrier()
        for i in range(npt // zr):
            pltpu.sync_copy(acc_sh.at[pl.ds(base + i * zr, zr)],
                            out_hbm.at[c, pl.ds(base + i * zr, zr)])

    return k


def _h2_body(x_ref, w1_ref, m1_ref, deg_ref, h2_ref):
    wm = w1_ref[...] * m1_ref[...]
    hh = jnp.dot(x_ref[...], wm, preferred_element_type=jnp.float32)
    d = deg_ref[:, 0] + deg_ref[:, 1] + 1.0
    a = lax.rsqrt(d)
    h2_ref[...] = hh * a[:, None]


def _final_body(acc_ref, h2_ref, deg_ref, b1_ref, w2_ref, m2_ref, b2_ref,
                out_ref):
    d = deg_ref[:, 0] + deg_ref[:, 1] + 1.0
    a = lax.rsqrt(d)
    tot = acc_ref[0] + acc_ref[1] + h2_ref[...]
    agg = tot * a[:, None] + b1_ref[0][None, :]
    hr = jnp.maximum(agg, 0.0)
    out_ref[...] = (jnp.dot(hr, w2_ref[...] * m2_ref[...],
                            preferred_element_type=jnp.float32)
                    + b2_ref[0][None, :])


def kernel(x, edge_index, edge_weight, W1, W1_mask, b1, W2, W2_mask, b2):
    n, f_in = x.shape
    h = W1.shape[1]
    c_out = W2.shape[1]
    e = edge_weight.shape[0]

    n_pad = ((n + NS * 16 - 1) // (NS * 16)) * (NS * 16)
    epw = (e + NW - 1) // NW
    nch = (epw + CH - 1) // CH
    e_pad = NW * nch * CH

    src = edge_index[0].astype(jnp.int32)
    dst = edge_index[1].astype(jnp.int32)
    pad = e_pad - e
    src_p = jnp.concatenate([src, jnp.zeros((pad,), jnp.int32)])
    dst_p = jnp.concatenate([dst, jnp.zeros((pad,), jnp.int32)])
    ew_p = jnp.concatenate([edge_weight, jnp.zeros((pad,), jnp.float32)])
    src3 = src_p.reshape(NW, nch, CH)
    dst3 = dst_p.reshape(NW, nch, CH)
    ew3 = ew_p.reshape(NW, nch, CH)
    ew2 = ew_p.reshape(NW, nch * CH)

    deg2 = _deg_kernel(n_pad, nch)(dst3, ew3)[:, :n].T

    rb = 1000
    grid = n // rb
    h2 = pl.pallas_call(
        _h2_body,
        grid=(grid,),
        in_specs=[
            pl.BlockSpec((rb, f_in), lambda j: (j, 0)),
            pl.BlockSpec((f_in, h), lambda j: (0, 0)),
            pl.BlockSpec((f_in, h), lambda j: (0, 0)),
            pl.BlockSpec((rb, NC), lambda j: (j, 0)),
        ],
        out_specs=pl.BlockSpec((rb, h), lambda j: (j, 0)),
        out_shape=jax.ShapeDtypeStruct((n, h), jnp.float32),
    )(x, W1, W1_mask, deg2)

    acc = _msg_kernel(n, n_pad, nch, h)(src3, dst3, ew2, h2)[:, :n]

    logits = pl.pallas_call(
        _final_body,
        grid=(grid,),
        in_specs=[
            pl.BlockSpec((NC, rb, h), lambda j: (0, j, 0)),
            pl.BlockSpec((rb, h), lambda j: (j, 0)),
            pl.BlockSpec((rb, NC), lambda j: (j, 0)),
            pl.BlockSpec((1, h), lambda j: (0, 0)),
            pl.BlockSpec((h, c_out), lambda j: (0, 0)),
            pl.BlockSpec((h, c_out), lambda j: (0, 0)),
            pl.BlockSpec((1, c_out), lambda j: (0, 0)),
        ],
        out_specs=pl.BlockSpec((rb, c_out), lambda j: (j, 0)),
        out_shape=jax.ShapeDtypeStruct((n, c_out), jnp.float32),
    )(acc, h2, deg2, b1.reshape(1, h), W2, W2_mask, b2.reshape(1, c_out))

    return logits


# speedup vs baseline: 28.9676x; 1.3718x over previous
import functools

import jax
import jax.numpy as jnp
from jax import lax
from jax.experimental import pallas as pl
from jax.experimental.pallas import tpu as pltpu
from jax.experimental.pallas import tpu_sc as plsc

NC = 2
NS = 16
NW = NC * NS
CH = 128


def _deg_kernel(n_pad, nch):
    npt = n_pad // NS

    @functools.partial(
        pl.kernel,
        out_type=jax.ShapeDtypeStruct((NC, n_pad), jnp.float32),
        mesh=plsc.VectorSubcoreMesh(core_axis_name="c", subcore_axis_name="s"),
        scratch_types=[
            pltpu.VMEM((nch, CH), jnp.int32),
            pltpu.VMEM((nch, CH), jnp.float32),
            pltpu.VMEM((npt,), jnp.float32),
            pltpu.VMEM_SHARED((n_pad,), jnp.float32),
        ],
    )
    def k(dst_hbm, ew_hbm, out_hbm, dstv, ewv, zb, deg_sh):
        c = lax.axis_index("c")
        s = lax.axis_index("s")
        w = c * NS + s

        def zfill(i, _):
            zb[pl.ds(i * 16, 16)] = jnp.zeros((16,), jnp.float32)
            return 0

        lax.fori_loop(0, npt // 16, zfill, 0)
        pltpu.sync_copy(zb, deg_sh.at[pl.ds(s * npt, npt)])
        pltpu.sync_copy(dst_hbm.at[w], dstv)
        pltpu.sync_copy(ew_hbm.at[w], ewv)
        plsc.subcore_barrier()

        def chunk(j, _):
            pltpu.sync_copy(ewv.at[j], deg_sh.at[dstv.at[j]], add=True)
            return 0

        lax.fori_loop(0, nch, chunk, 0)
        plsc.subcore_barrier()
        pltpu.sync_copy(deg_sh.at[pl.ds(s * npt, npt)],
                        out_hbm.at[c, pl.ds(s * npt, npt)])

    return k


def _msg_kernel(n, n_pad, nch, h):
    npt = n_pad // NS
    zr = 64
    nbuf = 3

    @functools.partial(
        pl.kernel,
        out_type=jax.ShapeDtypeStruct((NC, n_pad, h), jnp.float32),
        mesh=plsc.VectorSubcoreMesh(core_axis_name="c", subcore_axis_name="s"),
        scratch_types=[
            pltpu.VMEM((nch, CH), jnp.int32),
            pltpu.VMEM((nch, CH), jnp.int32),
            pltpu.VMEM((nch * CH,), jnp.float32),
            pltpu.VMEM((nbuf, CH, h), jnp.float32),
            pltpu.VMEM((zr, h), jnp.float32),
            pltpu.VMEM_SHARED((n_pad, h), jnp.float32),
            [pltpu.SemaphoreType.DMA] * nbuf,
            [pltpu.SemaphoreType.DMA] * nbuf,
        ],
        compiler_params=pltpu.CompilerParams(needs_layout_passes=False,
                                             use_tc_tiling_on_sc=False),
    )
    def k(src_hbm, dst_hbm, ew_hbm, h2_hbm, out_hbm,
          srcv, dstv, ewv, rowsv, zb, acc_sh, gsems, ssems):
        c = lax.axis_index("c")
        s = lax.axis_index("s")
        w = c * NS + s
        base = s * npt

        def zfill(i, _):
            for q in range(h // 16):
                zb[i, pl.ds(q * 16, 16)] = jnp.zeros((16,), jnp.float32)
            return 0

        lax.fori_loop(0, zr, zfill, 0)
        for i in range(npt // zr):
            pltpu.sync_copy(zb, acc_sh.at[pl.ds(base + i * zr, zr)])
        pltpu.sync_copy(src_hbm.at[w], srcv)
        pltpu.sync_copy(dst_hbm.at[w], dstv)
        pltpu.sync_copy(ew_hbm.at[w], ewv)
        plsc.subcore_barrier()

        def issue_gather(j, b):
            pltpu.async_copy(h2_hbm.at[srcv.at[j]], rowsv.at[b], gsems[b])

        def wait_gather(j, b):
            pltpu.make_async_copy(h2_hbm.at[srcv.at[j]], rowsv.at[b],
                                  gsems[b]).wait()

        def issue_scatter(j, b):
            pltpu.async_copy(rowsv.at[b], acc_sh.at[dstv.at[j]], ssems[b],
                             add=True)

        def wait_scatter(j, b):
            pltpu.make_async_copy(rowsv.at[b], acc_sh.at[dstv.at[j]],
                                  ssems[b]).wait()

        def scale(j, b):
            def grp(g, _):
                for kk in range(16):
                    le = g * 16 + kk
                    bi = jnp.full((16,), j * CH + le, jnp.int32)
                    wv = plsc.load_gather(ewv, [bi])
                    for q in range(h // 16):
                        sl = pl.ds(q * 16, 16)
                        rowsv[b, le, sl] = rowsv[b, le, sl] * wv
                return 0

            lax.fori_loop(0, CH // 16, grp, 0)

        issue_gather(0, 0)
        issue_gather(1, 1)
        wait_gather(0, 0)
        scale(0, 0)
        issue_scatter(0, 0)
        issue_gather(2, 2)
        wait_gather(1, 1)
        scale(1, 1)
        issue_scatter(1, 1)
        wait_scatter(0, 0)
        issue_gather(3, 0)

        nmain = nch - 2 - ((nch - 2) % nbuf)

        def body(it, _):
            jj = 2 + it * nbuf
            for db in range(nbuf):
                j = jj + db
                b = (2 + db) % nbuf
                wait_gather(j, b)
                scale(j, b)
                issue_scatter(j, b)
                bn = (b + 2) % nbuf
                wait_scatter(j - 1, bn)
                issue_gather(j + 2, bn)
            return 0

        lax.fori_loop(0, nmain // nbuf, body, 0)
        for j in range(nmain + 2, nch):
            b = j % nbuf
            wait_gather(j, b)
            scale(j, b)
            issue_scatter(j, b)
        for j in range(nmain + 1, nch):
            wait_scatter(j, j % nbuf)

        plsc.subcore_barrier()
        for i in range(npt // zr):
            pltpu.sync_copy(acc_sh.at[pl.ds(base + i * zr, zr)],
                            out_hbm.at[c, pl.ds(base + i * zr, zr)])

    return k


def _h2_body(x_ref, w1_ref, m1_ref, deg_ref, h2_ref):
    wm = w1_ref[...] * m1_ref[...]
    hh = jnp.dot(x_ref[...], wm, preferred_element_type=jnp.float32)
    d = deg_ref[:, 0] + deg_ref[:, 1] + 1.0
    a = lax.rsqrt(d)
    h2_ref[...] = hh * a[:, None]


def _final_body(acc_ref, h2_ref, deg_ref, b1_ref, w2_ref, m2_ref, b2_ref,
                out_ref):
    d = deg_ref[:, 0] + deg_ref[:, 1] + 1.0
    a = lax.rsqrt(d)
    tot = acc_ref[0] + acc_ref[1] + h2_ref[...]
    agg = tot * a[:, None] + b1_ref[0][None, :]
    hr = jnp.maximum(agg, 0.0)
    out_ref[...] = (jnp.dot(hr, w2_ref[...] * m2_ref[...],
                            preferred_element_type=jnp.float32)
                    + b2_ref[0][None, :])


def kernel(x, edge_index, edge_weight, W1, W1_mask, b1, W2, W2_mask, b2):
    n, f_in = x.shape
    h = W1.shape[1]
    c_out = W2.shape[1]
    e = edge_weight.shape[0]

    n_pad = ((n + NS * 16 - 1) // (NS * 16)) * (NS * 16)
    epw = (e + NW - 1) // NW
    nch = (epw + CH - 1) // CH
    e_pad = NW * nch * CH

    src = edge_index[0].astype(jnp.int32)
    dst = edge_index[1].astype(jnp.int32)
    pad = e_pad - e
    src_p = jnp.concatenate([src, jnp.zeros((pad,), jnp.int32)])
    dst_p = jnp.concatenate([dst, jnp.zeros((pad,), jnp.int32)])
    ew_p = jnp.concatenate([edge_weight, jnp.zeros((pad,), jnp.float32)])
    src3 = src_p.reshape(NW, nch, CH)
    dst3 = dst_p.reshape(NW, nch, CH)
    ew3 = ew_p.reshape(NW, nch, CH)
    ew2 = ew_p.reshape(NW, nch * CH)

    deg2 = _deg_kernel(n_pad, nch)(dst3, ew3)[:, :n].T

    rb = 1000
    grid = n // rb
    h2 = pl.pallas_call(
        _h2_body,
        grid=(grid,),
        in_specs=[
            pl.BlockSpec((rb, f_in), lambda j: (j, 0)),
            pl.BlockSpec((f_in, h), lambda j: (0, 0)),
            pl.BlockSpec((f_in, h), lambda j: (0, 0)),
            pl.BlockSpec((rb, NC), lambda j: (j, 0)),
        ],
        out_specs=pl.BlockSpec((rb, h), lambda j: (j, 0)),
        out_shape=jax.ShapeDtypeStruct((n, h), jnp.float32),
    )(x, W1, W1_mask, deg2)

    acc = _msg_kernel(n, n_pad, nch, h)(src3, dst3, ew2, h2)[:, :n]

    logits = pl.pallas_call(
        _final_body,
        grid=(grid,),
        in_specs=[
            pl.BlockSpec((NC, rb, h), lambda j: (0, j, 0)),
            pl.BlockSpec((rb, h), lambda j: (j, 0)),
            pl.BlockSpec((rb, NC), lambda j: (j, 0)),
            pl.BlockSpec((1, h), lambda j: (0, 0)),
            pl.BlockSpec((h, c_out), lambda j: (0, 0)),
            pl.BlockSpec((h, c_out), lambda j: (0, 0)),
            pl.BlockSpec((1, c_out), lambda j: (0, 0)),
        ],
        out_specs=pl.BlockSpec((rb, c_out), lambda j: (j, 0)),
        out_shape=jax.ShapeDtypeStruct((n, c_out), jnp.float32),
    )(acc, h2, deg2, b1.reshape(1, h), W2, W2_mask, b2.reshape(1, c_out))

    return logits
